# baseline (speedup 1.0000x reference)
import functools

import jax
import jax.numpy as jnp
from jax import lax
from jax.experimental import pallas as pl
from jax.experimental.pallas import tpu as pltpu
from jax.experimental.pallas import tpu_sc as plsc

VOCAB = 60000
D = 128
B = 4096
TPD = 200
NC = 2
NS = 16
NW = NC * NS
BAGS_W = B // NW
TOK_W = BAGS_W * TPD
LANES = 16
NG = D // (2 * LANES)
SPLIT = 128
REST = TPD - SPLIT
INV = 1.0 / TPD
NBUF = 4


def _emb_body(tok_hbm, w_hbm, out_hbm, idx_v, rows0, rows1, rows2, rows3,
              acc_v, sem0, sem1, sem2, sem3):
    bufs = (rows0, rows1, rows2, rows3)
    sems = (sem0, sem1, sem2, sem3)
    wid = lax.axis_index("s") * NC + lax.axis_index("c")
    tok_base = pl.multiple_of(wid * TOK_W, 8)
    pltpu.sync_copy(tok_hbm.at[pl.ds(tok_base, TOK_W)], idx_v)

    def descs(bag, buf, sem):
        off = pl.multiple_of(bag * TPD, 8)
        d0 = pltpu.make_async_copy(
            w_hbm.at[idx_v.at[pl.ds(off, SPLIT)]], buf.at[pl.ds(0, SPLIT)], sem)
        d1 = pltpu.make_async_copy(
            w_hbm.at[idx_v.at[pl.ds(off + SPLIT, REST)]],
            buf.at[pl.ds(SPLIT, REST)], sem)
        return d0, d1

    def start(bag, buf, sem):
        for d in descs(bag, buf, sem):
            d.start()

    def wait(bag, buf, sem):
        for d in descs(bag, buf, sem):
            d.wait()

    def reduce(bag, buf):
        hi_mask = jnp.full((LANES,), -65536, jnp.int32)

        def body(t, accs):
            out = list(accs)
            for g in range(NG):
                w = buf[t, pl.ds(LANES * g, LANES)]
                a = lax.bitcast_convert_type(lax.shift_left(w, 16), jnp.float32)
                b = lax.bitcast_convert_type(lax.bitwise_and(w, hi_mask), jnp.float32)
                out[2 * g] = out[2 * g] + a
                out[2 * g + 1] = out[2 * g + 1] + b
            return tuple(out)

        accs = lax.fori_loop(
            0, TPD, body,
            tuple(jnp.zeros((LANES,), jnp.float32) for _ in range(2 * NG)),
            unroll=4)
        for c in range(2 * NG):
            acc_v[bag, pl.ds(LANES * c, LANES)] = accs[c] * INV

    for j in range(NBUF):
        start(j, bufs[j], sems[j])

    def step(bag, j):
        wait(bag, bufs[j], sems[j])
        reduce(bag, bufs[j])

        @pl.when(bag + NBUF < BAGS_W)
        def _():
            start(bag + NBUF, bufs[j], sems[j])

    def outer(p, carry):
        for j in range(NBUF):
            step(NBUF * p + j, j)
        return carry

    lax.fori_loop(0, BAGS_W // NBUF, outer, 0)
    pltpu.sync_copy(acc_v, out_hbm.at[pl.ds(wid * BAGS_W, BAGS_W)])


@functools.partial(jax.jit, donate_argnums=())
def _emb_bag(flattened_tokens, weight):
    mesh = plsc.VectorSubcoreMesh(core_axis_name="c", subcore_axis_name="s")
    wb = lax.bitcast_convert_type(
        weight.astype(jnp.bfloat16).reshape(VOCAB, D // 2, 2), jnp.int32)
    out_k = pl.kernel(
        _emb_body,
        out_type=jax.ShapeDtypeStruct((B, D), jnp.float32),
        mesh=mesh,
        compiler_params=pltpu.CompilerParams(use_tc_tiling_on_sc=False),
        scratch_types=[
            pltpu.VMEM((TOK_W,), jnp.int32),
            pltpu.VMEM((TPD, D // 2), jnp.int32),
            pltpu.VMEM((TPD, D // 2), jnp.int32),
            pltpu.VMEM((TPD, D // 2), jnp.int32),
            pltpu.VMEM((TPD, D // 2), jnp.int32),
            pltpu.VMEM((BAGS_W, D), jnp.float32),
            pltpu.SemaphoreType.DMA,
            pltpu.SemaphoreType.DMA,
            pltpu.SemaphoreType.DMA,
            pltpu.SemaphoreType.DMA,
        ],
    )(flattened_tokens, wb)
    return out_k.reshape(B, NG, 2, LANES).swapaxes(2, 3).reshape(B, D)


def kernel(flattened_tokens, offsets, weight):
    del offsets
    return _emb_bag(flattened_tokens, weight)


# speedup vs baseline: 1.9598x; 1.9598x over previous
import functools

import jax
import jax.numpy as jnp
from jax import lax
from jax.experimental import pallas as pl
from jax.experimental.pallas import tpu as pltpu
from jax.experimental.pallas import tpu_sc as plsc

VOCAB = 60000
D = 128
B = 4096
TPD = 200
NC = 2
NS = 16
NW = NC * NS
BAGS_W = B // NW
TOK_W = BAGS_W * TPD
LANES = 16
ND = D // LANES
SPLIT = 128
REST = TPD - SPLIT
INV = 1.0 / TPD
NBUF = 3


def _emb_body(tok_hbm, w_hbm, out_hbm, idx_v, rows0, rows1, rows2,
              acc_v, sem0, sem1, sem2):
    bufs = (rows0, rows1, rows2)
    sems = (sem0, sem1, sem2)
    wid = lax.axis_index("s") * NC + lax.axis_index("c")
    tok_base = pl.multiple_of(wid * TOK_W, 8)
    pltpu.sync_copy(tok_hbm.at[pl.ds(tok_base, TOK_W)], idx_v)

    def descs(bag, buf, sem):
        off = pl.multiple_of(bag * TPD, 8)
        d0 = pltpu.make_async_copy(
            w_hbm.at[idx_v.at[pl.ds(off, SPLIT)]], buf.at[pl.ds(0, SPLIT)], sem)
        d1 = pltpu.make_async_copy(
            w_hbm.at[idx_v.at[pl.ds(off + SPLIT, REST)]],
            buf.at[pl.ds(SPLIT, REST)], sem)
        return d0, d1

    def start(bag, buf, sem):
        for d in descs(bag, buf, sem):
            d.start()

    def wait(bag, buf, sem):
        for d in descs(bag, buf, sem):
            d.wait()

    def reduce(bag, buf):
        def body(t, accs):
            return tuple(accs[d] + buf[t, pl.ds(LANES * d, LANES)]
                         for d in range(ND))

        accs = lax.fori_loop(
            0, TPD, body,
            tuple(jnp.zeros((LANES,), jnp.float32) for _ in range(ND)),
            unroll=4)
        for c in range(ND):
            acc_v[bag, pl.ds(LANES * c, LANES)] = accs[c] * INV

    for j in range(NBUF):
        start(j, bufs[j], sems[j])

    def step(bag, j):
        wait(bag, bufs[j], sems[j])
        reduce(bag, bufs[j])

        @pl.when(bag + NBUF < BAGS_W)
        def _():
            start(bag + NBUF, bufs[j], sems[j])

    def outer(p, carry):
        for j in range(NBUF):
            step(NBUF * p + j, j)
        return carry

    full = BAGS_W // NBUF
    lax.fori_loop(0, full, outer, 0)
    for j in range(BAGS_W - NBUF * full):
        step(NBUF * full + j, j)

    pltpu.sync_copy(acc_v, out_hbm.at[pl.ds(wid * BAGS_W, BAGS_W)])


@functools.partial(jax.jit, donate_argnums=())
def _emb_bag(flattened_tokens, weight):
    mesh = plsc.VectorSubcoreMesh(core_axis_name="c", subcore_axis_name="s")
    out_k = pl.kernel(
        _emb_body,
        out_type=jax.ShapeDtypeStruct((B, D), jnp.float32),
        mesh=mesh,
        scratch_types=[
            pltpu.VMEM((TOK_W,), jnp.int32),
            pltpu.VMEM((TPD, D), jnp.float32),
            pltpu.VMEM((TPD, D), jnp.float32),
            pltpu.VMEM((TPD, D), jnp.float32),
            pltpu.VMEM((BAGS_W, D), jnp.float32),
            pltpu.SemaphoreType.DMA,
            pltpu.SemaphoreType.DMA,
            pltpu.SemaphoreType.DMA,
        ],
    )(flattened_tokens, weight)
    return out_k


def kernel(flattened_tokens, offsets, weight):
    del offsets
    return _emb_bag(flattened_tokens, weight)
